# baseline (speedup 1.0000x reference)
import functools

import jax
import jax.numpy as jnp
from jax import lax
from jax.experimental import pallas as pl
from jax.experimental.pallas import tpu as pltpu
from jax.experimental.pallas import tpu_sc as plsc

VOCAB = 1000000
EMBED = 64
CTX = 200
HIDDEN = 128

TILE_V = 25088
NSTR = 2
NTILES = (VOCAB + TILE_V - 1) // TILE_V
NSTEPS = (NTILES + NSTR - 1) // NSTR
STEP_V = NSTR * TILE_V

TILE_F = 65536
NFIX = (VOCAB + TILE_F - 1) // TILE_F


NACT = 25
PER = 8


def _gather_sum_sc(inputs, embT):
    mesh = plsc.VectorSubcoreMesh(core_axis_name="c", subcore_axis_name="s")

    @functools.partial(
        pl.kernel,
        mesh=mesh,
        compiler_params=pltpu.CompilerParams(needs_layout_passes=False),
        out_type=jax.ShapeDtypeStruct((32, EMBED), jnp.float32),
        scratch_types=[
            pltpu.VMEM((16,), jnp.int32),
            pltpu.VMEM((PER, EMBED, 128), jnp.float32),
            pltpu.VMEM((1, EMBED), jnp.float32),
            pltpu.SemaphoreType.DMA,
        ],
    )
    def k(idx_hbm, embT_hbm, out_hbm, idx_v, blk_v, acc_v, sem):
        c = lax.axis_index("c")
        s = lax.axis_index("s")
        w = c * 16 + s
        active = w < NACT

        for g in range(4):
            acc_v[0, pl.ds(16 * g, 16)] = jnp.zeros((16,), jnp.float32)

        @pl.when(active)
        def _():
            pltpu.sync_copy(idx_hbm.at[pl.ds(PER * w, PER)],
                            idx_v.at[pl.ds(0, PER)])
            vec = idx_v[...]
            for j in range(PER):
                col128 = pl.multiple_of((vec[j] // 128) * 128, 128)
                pltpu.make_async_copy(
                    embT_hbm.at[:, pl.ds(col128, 128)], blk_v.at[j], sem
                ).start()
            for j in range(PER):
                pltpu.make_async_copy(
                    embT_hbm.at[:, pl.ds(0, 128)], blk_v.at[j], sem
                ).wait()
            accs = [jnp.zeros((16,), jnp.float32) for _ in range(4)]
            for j in range(PER):
                lane = jnp.full((16,), vec[j] % 128, jnp.int32)
                for g in range(4):
                    rows = lax.iota(jnp.int32, 16) + 16 * g
                    accs[g] = accs[g] + plsc.load_gather(
                        blk_v.at[j], [rows, lane]
                    )
            for g in range(4):
                acc_v[0, pl.ds(16 * g, 16)] = accs[g]

        pltpu.sync_copy(acc_v, out_hbm.at[pl.ds(w, 1)])

    return k(inputs, embT)


def _proj_body(e_ref, w1t_ref, b1_ref, *rest):
    w2_refs = rest[:NSTR]
    b2_ref, out_ref, lse_ref, h_s, m_s, s_s = rest[NSTR:]
    i = pl.program_id(0)

    @pl.when(i == 0)
    def _():
        e = jnp.sum(e_ref[...], axis=0, keepdims=True)
        h = jnp.dot(e, w1t_ref[...],
                    preferred_element_type=jnp.float32) + b1_ref[...]
        h_s[...] = jnp.maximum(h, 0.0)
        m_s[0] = -jnp.inf
        s_s[...] = jnp.zeros((1, 128), jnp.float32)

    h = h_s[...]
    dn = (((1,), (1,)), ((), ()))
    ts = [lax.dot_general(h, w_ref[...], dn,
                          preferred_element_type=jnp.float32)
          for w_ref in w2_refs]
    t = jnp.concatenate(ts, axis=1) + b2_ref[...].reshape(1, STEP_V)
    lanes = lax.broadcasted_iota(jnp.int32, (1, STEP_V), 1)
    valid = (i * STEP_V + lanes) < VOCAB
    t = jnp.where(valid, t, -jnp.inf)
    out_ref[...] = t

    m_old = m_s[0]
    m_new = jnp.maximum(m_old, jnp.max(t))
    m_s[0] = m_new
    corr = jnp.exp(jnp.full((1, 128), m_old - m_new, jnp.float32))
    tsum = jnp.sum(jnp.exp(t - m_new))
    s_s[...] = s_s[...] * corr + jnp.full((1, 128), tsum, jnp.float32)

    @pl.when(i == NSTEPS - 1)
    def _():
        lse_ref[...] = m_s[0] + jnp.log(s_s[...])


def _projection_tc(e, w1t, b1, w2, b2):
    return pl.pallas_call(
        _proj_body,
        grid=(NSTEPS,),
        in_specs=[
            pl.BlockSpec((32, EMBED), lambda i: (0, 0)),
            pl.BlockSpec((EMBED, HIDDEN), lambda i: (0, 0)),
            pl.BlockSpec((1, HIDDEN), lambda i: (0, 0)),
        ] + [
            pl.BlockSpec((TILE_V, HIDDEN),
                         functools.partial(lambda j, i: (NSTR * i + j, 0), j))
            for j in range(NSTR)
        ] + [
            pl.BlockSpec((STEP_V,), lambda i: (i,)),
        ],
        out_specs=[
            pl.BlockSpec((1, STEP_V), lambda i: (0, i)),
            pl.BlockSpec((1, 128), lambda i: (0, 0)),
        ],
        scratch_shapes=[
            pltpu.VMEM((1, HIDDEN), jnp.float32),
            pltpu.SMEM((1,), jnp.float32),
            pltpu.VMEM((1, 128), jnp.float32),
        ],
        out_shape=[
            jax.ShapeDtypeStruct((1, VOCAB), jnp.float32),
            jax.ShapeDtypeStruct((1, 128), jnp.float32),
        ],
    )(e, w1t, b1, *([w2] * NSTR), b2)


def _fix_body(logits_ref, lse_ref, out_ref):
    out_ref[...] = logits_ref[...] - lse_ref[0]


def _fixup_tc(logits, lse):
    return pl.pallas_call(
        _fix_body,
        grid=(NFIX,),
        in_specs=[
            pl.BlockSpec((1, TILE_F), lambda i: (0, i)),
            pl.BlockSpec(memory_space=pltpu.SMEM),
        ],
        out_specs=pl.BlockSpec((1, TILE_F), lambda i: (0, i)),
        out_shape=jax.ShapeDtypeStruct((1, VOCAB), jnp.float32),
        input_output_aliases={0: 0},
    )(logits, lse)


def kernel(inputs, emb, W1, b1, W2, b2):
    e2 = _gather_sum_sc(inputs.astype(jnp.int32), emb.T)
    w1t = W1.T
    logits, lse = _projection_tc(e2, w1t, b1.reshape(1, HIDDEN), W2, b2)
    lse_s = lse[:1, :1].reshape(1)
    return _fixup_tc(logits, lse_s)


# speedup vs baseline: 1.0120x; 1.0120x over previous
import functools

import jax
import jax.numpy as jnp
from jax import lax
from jax.experimental import pallas as pl
from jax.experimental.pallas import tpu as pltpu
from jax.experimental.pallas import tpu_sc as plsc

VOCAB = 1000000
EMBED = 64
CTX = 200
HIDDEN = 128

TILE_V = 16384
NSTR = 2
NTILES = (VOCAB + TILE_V - 1) // TILE_V
NSTEPS = (NTILES + NSTR - 1) // NSTR
STEP_V = NSTR * TILE_V

TILE_F = 65536
NFIX = (VOCAB + TILE_F - 1) // TILE_F


NACT = 25
PER = 8


def _gather_sum_sc(inputs, embT):
    mesh = plsc.VectorSubcoreMesh(core_axis_name="c", subcore_axis_name="s")

    @functools.partial(
        pl.kernel,
        mesh=mesh,
        compiler_params=pltpu.CompilerParams(needs_layout_passes=False),
        out_type=jax.ShapeDtypeStruct((32, EMBED), jnp.float32),
        scratch_types=[
            pltpu.VMEM((16,), jnp.int32),
            pltpu.VMEM((PER, EMBED, 128), jnp.float32),
            pltpu.VMEM((1, EMBED), jnp.float32),
            pltpu.SemaphoreType.DMA,
        ],
    )
    def k(idx_hbm, embT_hbm, out_hbm, idx_v, blk_v, acc_v, sem):
        c = lax.axis_index("c")
        s = lax.axis_index("s")
        w = c * 16 + s
        active = w < NACT

        for g in range(4):
            acc_v[0, pl.ds(16 * g, 16)] = jnp.zeros((16,), jnp.float32)

        @pl.when(active)
        def _():
            pltpu.sync_copy(idx_hbm.at[pl.ds(PER * w, PER)],
                            idx_v.at[pl.ds(0, PER)])
            vec = idx_v[...]
            for j in range(PER):
                col128 = pl.multiple_of((vec[j] // 128) * 128, 128)
                pltpu.make_async_copy(
                    embT_hbm.at[:, pl.ds(col128, 128)], blk_v.at[j], sem
                ).start()
            for j in range(PER):
                pltpu.make_async_copy(
                    embT_hbm.at[:, pl.ds(0, 128)], blk_v.at[j], sem
                ).wait()
            accs = [jnp.zeros((16,), jnp.float32) for _ in range(4)]
            for j in range(PER):
                lane = jnp.full((16,), vec[j] % 128, jnp.int32)
                for g in range(4):
                    rows = lax.iota(jnp.int32, 16) + 16 * g
                    accs[g] = accs[g] + plsc.load_gather(
                        blk_v.at[j], [rows, lane]
                    )
            for g in range(4):
                acc_v[0, pl.ds(16 * g, 16)] = accs[g]

        pltpu.sync_copy(acc_v, out_hbm.at[pl.ds(w, 1)])

    return k(inputs, embT)


def _proj_body(e_ref, w1t_ref, b1_ref, *rest):
    w2_refs = rest[:NSTR]
    b2_ref, out_ref, lse_ref, h_s, m_s, s_s = rest[NSTR:]
    i = pl.program_id(0)

    @pl.when(i == 0)
    def _():
        e = jnp.sum(e_ref[...], axis=0, keepdims=True)
        h = jnp.dot(e, w1t_ref[...],
                    preferred_element_type=jnp.float32) + b1_ref[...]
        h_s[...] = jnp.maximum(h, 0.0)
        m_s[0] = -jnp.inf
        s_s[...] = jnp.zeros((1, 128), jnp.float32)

    h = h_s[...]
    dn = (((1,), (1,)), ((), ()))
    ts = [lax.dot_general(h, w_ref[...], dn,
                          preferred_element_type=jnp.float32)
          for w_ref in w2_refs]
    t = jnp.concatenate(ts, axis=1) + b2_ref[...].reshape(1, STEP_V)
    lanes = lax.broadcasted_iota(jnp.int32, (1, STEP_V), 1)
    valid = (i * STEP_V + lanes) < VOCAB
    t = jnp.where(valid, t, -jnp.inf)
    out_ref[...] = t

    m_old = m_s[0]
    m_new = jnp.maximum(m_old, jnp.max(t))
    m_s[0] = m_new
    corr = jnp.exp(jnp.full((1, 128), m_old - m_new, jnp.float32))
    tsum = jnp.sum(jnp.exp(t - m_new))
    s_s[...] = s_s[...] * corr + jnp.full((1, 128), tsum, jnp.float32)

    @pl.when(i == NSTEPS - 1)
    def _():
        lse_ref[...] = m_s[0] + jnp.log(s_s[...])


def _projection_tc(e, w1t, b1, w2, b2):
    return pl.pallas_call(
        _proj_body,
        grid=(NSTEPS,),
        in_specs=[
            pl.BlockSpec((32, EMBED), lambda i: (0, 0)),
            pl.BlockSpec((EMBED, HIDDEN), lambda i: (0, 0)),
            pl.BlockSpec((1, HIDDEN), lambda i: (0, 0)),
        ] + [
            pl.BlockSpec((TILE_V, HIDDEN),
                         functools.partial(lambda j, i: (NSTR * i + j, 0), j))
            for j in range(NSTR)
        ] + [
            pl.BlockSpec((STEP_V,), lambda i: (i,)),
        ],
        out_specs=[
            pl.BlockSpec((1, STEP_V), lambda i: (0, i)),
            pl.BlockSpec((1, 128), lambda i: (0, 0)),
        ],
        scratch_shapes=[
            pltpu.VMEM((1, HIDDEN), jnp.float32),
            pltpu.SMEM((1,), jnp.float32),
            pltpu.VMEM((1, 128), jnp.float32),
        ],
        out_shape=[
            jax.ShapeDtypeStruct((1, VOCAB), jnp.float32),
            jax.ShapeDtypeStruct((1, 128), jnp.float32),
        ],
    )(e, w1t, b1, *([w2] * NSTR), b2)


def _fix_body(logits_ref, lse_ref, out_ref):
    out_ref[...] = logits_ref[...] - lse_ref[0]


def _fixup_tc(logits, lse):
    return pl.pallas_call(
        _fix_body,
        grid=(NFIX,),
        in_specs=[
            pl.BlockSpec((1, TILE_F), lambda i: (0, i)),
            pl.BlockSpec(memory_space=pltpu.SMEM),
        ],
        out_specs=pl.BlockSpec((1, TILE_F), lambda i: (0, i)),
        out_shape=jax.ShapeDtypeStruct((1, VOCAB), jnp.float32),
        input_output_aliases={0: 0},
    )(logits, lse)


def kernel(inputs, emb, W1, b1, W2, b2):
    e2 = _gather_sum_sc(inputs.astype(jnp.int32), emb.T)
    w1t = W1.T
    logits, lse = _projection_tc(e2, w1t, b1.reshape(1, HIDDEN), W2, b2)
    lse_s = lse[:1, :1].reshape(1)
    return _fixup_tc(logits, lse_s)
